# Initial kernel scaffold; baseline (speedup 1.0000x reference)
import functools

import jax
import jax.numpy as jnp
from jax import lax
from jax.experimental import pallas as pl
from jax.experimental.pallas import tpu as pltpu
from jax.experimental.pallas import tpu_sc as plsc

_N = 10000
_D = 128
_E = 320000
_NCLS = 40

_NC = 2
_NS = 16
_NW = _NC * _NS
_K = 128
_CW = 79
_EPAD = _NW * _CW * _K
_CD = _EPAD // (_NS * _K)
_NPAD = 10240
_RPT = _NPAD // _NS
_HB = _RPT // 2


def _mp_body(tab_h, src_h, dst_h, out_h,
             agg_sh, deg_sh, src_v, dst_v, rows_v, ones_v, zbuf, degb, sem):
    c = lax.axis_index("c")
    s = lax.axis_index("s")
    wid = s * _NC + c
    r0 = s * _RPT

    def _init_zrow(i, carry):
        for j in range(_D // 16):
            zbuf[i, pl.ds(j * 16, 16)] = jnp.zeros((16,), jnp.float32)
        return carry
    lax.fori_loop(0, _HB, _init_zrow, 0)

    def _init_ones(j, carry):
        ones_v[pl.ds(j * 16, 16)] = jnp.ones((16,), jnp.float32)
        return carry
    lax.fori_loop(0, _K // 16, _init_ones, 0)

    def _init_degb(j, carry):
        degb[pl.ds(j * 16, 16)] = jnp.zeros((16,), jnp.float32)
        return carry
    lax.fori_loop(0, _RPT // 16, _init_degb, 0)

    pltpu.sync_copy(zbuf, agg_sh.at[pl.ds(r0, _HB), :])
    pltpu.sync_copy(zbuf, agg_sh.at[pl.ds(r0 + _HB, _HB), :])
    pltpu.sync_copy(degb, deg_sh.at[pl.ds(r0, _RPT)])
    plsc.subcore_barrier()

    def _deg_step(t, carry):
        base = (s * _CD + t) * _K
        pltpu.sync_copy(dst_h.at[pl.ds(base, _K)], dst_v)
        pltpu.sync_copy(ones_v, deg_sh.at[dst_v], add=True)
        return carry
    lax.fori_loop(0, _CD, _deg_step, 0)

    def _agg_step(t, carry):
        base = (wid * _CW + t) * _K
        pltpu.sync_copy(src_h.at[pl.ds(base, _K)], src_v)
        pltpu.sync_copy(dst_h.at[pl.ds(base, _K)], dst_v)
        pltpu.async_copy(tab_h.at[src_v], rows_v, sem).wait()
        pltpu.sync_copy(rows_v, agg_sh.at[dst_v], add=True)
        return carry
    lax.fori_loop(0, _CW, _agg_step, 0)

    plsc.subcore_barrier()

    pltpu.sync_copy(deg_sh.at[pl.ds(r0, _RPT)], degb)
    for blk in range(2):
        rb = r0 + blk * _HB

        pltpu.sync_copy(agg_sh.at[pl.ds(rb, _HB), :], zbuf)

        def _norm_row(r, carry, blk=blk):
            d = degb[blk * _HB + r]
            dv = jnp.full((16,), d, jnp.float32)
            inv = 1.0 / jnp.maximum(dv, 1.0)
            for j in range(_D // 16):
                zbuf[r, pl.ds(j * 16, 16)] = zbuf[r, pl.ds(j * 16, 16)] * inv
            return carry
        lax.fori_loop(0, _HB, _norm_row, 0)

        pltpu.sync_copy(zbuf, out_h.at[c, pl.ds(rb, _HB), :])


_mp_call = functools.partial(
    pl.kernel,
    out_type=jax.ShapeDtypeStruct((_NC, _NPAD, _D), jnp.float32),
    mesh=plsc.VectorSubcoreMesh(core_axis_name="c", subcore_axis_name="s"),
    scratch_types=[
        pltpu.VMEM_SHARED((_NPAD, _D), jnp.float32),
        pltpu.VMEM_SHARED((_NPAD,), jnp.float32),
        pltpu.VMEM((_K,), jnp.int32),
        pltpu.VMEM((_K,), jnp.int32),
        pltpu.VMEM((_K, _D), jnp.float32),
        pltpu.VMEM((_K,), jnp.float32),
        pltpu.VMEM((_HB, _D), jnp.float32),
        pltpu.VMEM((_RPT,), jnp.float32),
        pltpu.SemaphoreType.DMA,
    ],
)(_mp_body)


def _tc1_body(p_ref, w_ref, b_ref, g_ref, be_ref, a_ref, o_ref):
    h = p_ref[0, :_N, :] + p_ref[1, :_N, :]
    h = jnp.dot(h, w_ref[...], preferred_element_type=jnp.float32) + b_ref[...]
    m = jnp.mean(h, axis=0, keepdims=True)
    v = jnp.mean((h - m) * (h - m), axis=0, keepdims=True)
    h = (h - m) * lax.rsqrt(v + 1e-5) * g_ref[...] + be_ref[...]
    a = a_ref[0, 0]
    o_ref[...] = jnp.where(h > 0, h, a * h)


_tc1_call = pl.pallas_call(
    _tc1_body,
    out_shape=jax.ShapeDtypeStruct((_N, _D), jnp.float32),
)


def _tc2_body(p_ref, w_ref, b_ref, g_ref, be_ref, a_ref, wc_ref, bc_ref,
              o_ref):
    h = p_ref[0, :_N, :] + p_ref[1, :_N, :]
    h = jnp.dot(h, w_ref[...], preferred_element_type=jnp.float32) + b_ref[...]
    m = jnp.mean(h, axis=0, keepdims=True)
    v = jnp.mean((h - m) * (h - m), axis=0, keepdims=True)
    h = (h - m) * lax.rsqrt(v + 1e-5) * g_ref[...] + be_ref[...]
    a = a_ref[0, 0]
    h = jnp.where(h > 0, h, a * h)
    lg = jnp.dot(h, wc_ref[...], preferred_element_type=jnp.float32)
    lg = lg + bc_ref[...]
    mx = jnp.max(lg, axis=-1, keepdims=True)
    e = jnp.exp(lg - mx)
    o_ref[...] = e / jnp.sum(e, axis=-1, keepdims=True) + 1e-10


_tc2_call = pl.pallas_call(
    _tc2_body,
    out_shape=jax.ShapeDtypeStruct((_N, _NCLS), jnp.float32),
)


def kernel(x, edge_index, W1, b1, g1, be1, a1, W2, b2, g2, be2, a2, Wc, bc):
    src = edge_index[0].astype(jnp.int32)
    dst = edge_index[1].astype(jnp.int32)
    pad = _EPAD - _E
    srcp = jnp.concatenate([src, jnp.zeros((pad,), jnp.int32)])
    dstp = jnp.concatenate([dst, jnp.full((pad,), _N, jnp.int32)])

    p1 = _mp_call(x, srcp, dstp)
    h1 = _tc1_call(p1, W1, b1.reshape(1, _D), g1.reshape(1, _D),
                   be1.reshape(1, _D), a1.reshape(1, 1))
    p2 = _mp_call(h1, srcp, dstp)
    return _tc2_call(p2, W2, b2.reshape(1, _D), g2.reshape(1, _D),
                     be2.reshape(1, _D), a2.reshape(1, 1),
                     Wc, bc.reshape(1, _NCLS))


# speedup vs baseline: 3.3967x; 3.3967x over previous
import functools

import jax
import jax.numpy as jnp
from jax import lax
from jax.experimental import pallas as pl
from jax.experimental.pallas import tpu as pltpu
from jax.experimental.pallas import tpu_sc as plsc

_N = 10000
_D = 128
_E = 320000
_NCLS = 40

_NC = 2
_NS = 16
_NW = _NC * _NS
_K = 128
_CW = 79
_EPAD = _NW * _CW * _K
_CD = _EPAD // (_NS * _K)
_NPAD = 10240
_RPT = _NPAD // _NS
_HB = _RPT // 4


def _mp_body(tab_h, src_h, dst_h, out_h,
             agg_sh, deg_sh, src_v, dst_v, rows_v, ones_v, zbuf, degb, sem):
    c = lax.axis_index("c")
    s = lax.axis_index("s")
    wid = s * _NC + c
    r0 = s * _RPT

    def _init_zrow(i, carry):
        for j in range(_D // 16):
            zbuf[i, pl.ds(j * 16, 16)] = jnp.zeros((16,), jnp.float32)
        return carry
    lax.fori_loop(0, _HB, _init_zrow, 0)

    def _init_ones(j, carry):
        ones_v[pl.ds(j * 16, 16)] = jnp.ones((16,), jnp.float32)
        return carry
    lax.fori_loop(0, _K // 16, _init_ones, 0)

    def _init_degb(j, carry):
        degb[pl.ds(j * 16, 16)] = jnp.zeros((16,), jnp.float32)
        return carry
    lax.fori_loop(0, _RPT // 16, _init_degb, 0)

    for blk in range(_RPT // _HB):
        pltpu.sync_copy(zbuf, agg_sh.at[pl.ds(r0 + blk * _HB, _HB), :])
    pltpu.sync_copy(degb, deg_sh.at[pl.ds(r0, _RPT)])
    plsc.subcore_barrier()

    def _deg_step(t, carry):
        base = (s * _CD + t) * _K
        pltpu.sync_copy(dst_h.at[pl.ds(base, _K)], dst_v)
        pltpu.sync_copy(ones_v, deg_sh.at[dst_v], add=True)
        return carry
    lax.fori_loop(0, _CD, _deg_step, 0)

    def _agg_step(t, carry):
        base = (wid * _CW + t) * _K
        pltpu.sync_copy(src_h.at[pl.ds(base, _K)], src_v)
        pltpu.sync_copy(dst_h.at[pl.ds(base, _K)], dst_v)
        pltpu.async_copy(tab_h.at[src_v], rows_v, sem).wait()
        pltpu.sync_copy(rows_v, agg_sh.at[dst_v], add=True)
        return carry
    lax.fori_loop(0, _CW, _agg_step, 0)

    plsc.subcore_barrier()

    pltpu.sync_copy(deg_sh.at[pl.ds(r0, _RPT)], degb)
    for blk in range(_RPT // _HB):
        rb = r0 + blk * _HB

        pltpu.sync_copy(agg_sh.at[pl.ds(rb, _HB), :], zbuf)

        def _norm_rows(t, carry, blk=blk):
            dv = degb[pl.ds(blk * _HB + t * 16, 16)]
            inv = 1.0 / jnp.maximum(dv, 1.0)
            for i in range(16):
                r = t * 16 + i
                iv = inv[i]
                for j in range(_D // 16):
                    zbuf[r, pl.ds(j * 16, 16)] = (
                        zbuf[r, pl.ds(j * 16, 16)] * iv)
            return carry
        lax.fori_loop(0, _HB // 16, _norm_rows, 0)

        pltpu.sync_copy(zbuf, out_h.at[c, pl.ds(rb, _HB), :])


_mp_call = functools.partial(
    pl.kernel,
    out_type=jax.ShapeDtypeStruct((_NC, _NPAD, _D), jnp.float32),
    mesh=plsc.VectorSubcoreMesh(core_axis_name="c", subcore_axis_name="s"),
    scratch_types=[
        pltpu.VMEM_SHARED((_NPAD, _D), jnp.float32),
        pltpu.VMEM_SHARED((_NPAD,), jnp.float32),
        pltpu.VMEM((_K,), jnp.int32),
        pltpu.VMEM((_K,), jnp.int32),
        pltpu.VMEM((_K, _D), jnp.float32),
        pltpu.VMEM((_K,), jnp.float32),
        pltpu.VMEM((_HB, _D), jnp.float32),
        pltpu.VMEM((_RPT,), jnp.float32),
        pltpu.SemaphoreType.DMA,
    ],
)(_mp_body)


def _tc1_body(p_ref, w_ref, b_ref, g_ref, be_ref, a_ref, o_ref):
    h = p_ref[0, :_N, :] + p_ref[1, :_N, :]
    h = jnp.dot(h, w_ref[...], preferred_element_type=jnp.float32) + b_ref[...]
    m = jnp.mean(h, axis=0, keepdims=True)
    v = jnp.mean((h - m) * (h - m), axis=0, keepdims=True)
    h = (h - m) * lax.rsqrt(v + 1e-5) * g_ref[...] + be_ref[...]
    a = a_ref[0, 0]
    o_ref[...] = jnp.where(h > 0, h, a * h)


_tc1_call = pl.pallas_call(
    _tc1_body,
    out_shape=jax.ShapeDtypeStruct((_N, _D), jnp.float32),
)


def _tc2_body(p_ref, w_ref, b_ref, g_ref, be_ref, a_ref, wc_ref, bc_ref,
              o_ref):
    h = p_ref[0, :_N, :] + p_ref[1, :_N, :]
    h = jnp.dot(h, w_ref[...], preferred_element_type=jnp.float32) + b_ref[...]
    m = jnp.mean(h, axis=0, keepdims=True)
    v = jnp.mean((h - m) * (h - m), axis=0, keepdims=True)
    h = (h - m) * lax.rsqrt(v + 1e-5) * g_ref[...] + be_ref[...]
    a = a_ref[0, 0]
    h = jnp.where(h > 0, h, a * h)
    lg = jnp.dot(h, wc_ref[...], preferred_element_type=jnp.float32)
    lg = lg + bc_ref[...]
    mx = jnp.max(lg, axis=-1, keepdims=True)
    e = jnp.exp(lg - mx)
    o_ref[...] = e / jnp.sum(e, axis=-1, keepdims=True) + 1e-10


_tc2_call = pl.pallas_call(
    _tc2_body,
    out_shape=jax.ShapeDtypeStruct((_N, _NCLS), jnp.float32),
)


def kernel(x, edge_index, W1, b1, g1, be1, a1, W2, b2, g2, be2, a2, Wc, bc):
    src = edge_index[0].astype(jnp.int32)
    dst = edge_index[1].astype(jnp.int32)
    pad = _EPAD - _E
    srcp = jnp.concatenate([src, jnp.zeros((pad,), jnp.int32)])
    dstp = jnp.concatenate([dst, jnp.full((pad,), _N, jnp.int32)])

    p1 = _mp_call(x, srcp, dstp)
    h1 = _tc1_call(p1, W1, b1.reshape(1, _D), g1.reshape(1, _D),
                   be1.reshape(1, _D), a1.reshape(1, 1))
    p2 = _mp_call(h1, srcp, dstp)
    return _tc2_call(p2, W2, b2.reshape(1, _D), g2.reshape(1, _D),
                     be2.reshape(1, _D), a2.reshape(1, 1),
                     Wc, bc.reshape(1, _NCLS))


# speedup vs baseline: 3.4383x; 1.0122x over previous
import jax
import jax.numpy as jnp
from jax import lax
from jax.experimental import pallas as pl
from jax.experimental.pallas import tpu as pltpu
from jax.experimental.pallas import tpu_sc as plsc

_N = 10000
_D = 128
_E = 320000
_NCLS = 40

_NC = 2
_NS = 16
_NW = _NC * _NS
_K = 128
_IB = 16
_CW = 80
_EPAD = _NW * _CW * _K
_CD = _EPAD // (_NS * _K)
_NPAD = 10240
_RPT = _NPAD // _NS
_HB = 64


def _zero_vec(ref, n):
    def _z(j, carry):
        ref[pl.ds(j * 16, 16)] = jnp.zeros((16,), jnp.float32)
        return carry
    lax.fori_loop(0, n // 16, _z, 0)


def _gather_scatter_edges(tab_h, src_h, dst_h, agg_sh,
                          srcbig, dstbig, rowsA, rowsB, semA, semB, wid):
    for bi in range(_CW // _IB):
        base = wid * _CW + bi * _IB
        pltpu.sync_copy(src_h.at[pl.ds(base, _IB), :], srcbig)
        pltpu.sync_copy(dst_h.at[pl.ds(base, _IB), :], dstbig)
        src2 = srcbig
        dst2 = dstbig
        pending = None
        for t in range(_IB):
            buf, sem = (rowsA, semA) if t % 2 == 0 else (rowsB, semB)
            cp = pltpu.async_copy(tab_h.at[src2.at[t]], buf, sem)
            if pending is not None:
                pending[0].wait()
                pltpu.sync_copy(pending[1], agg_sh.at[dst2.at[pending[2]]],
                                add=True)
            pending = (cp, buf, t)
        pending[0].wait()
        pltpu.sync_copy(pending[1], agg_sh.at[dst2.at[pending[2]]], add=True)


def _mp1_body(tab_h, src_h, dst_h, out_h, invdeg_h,
              agg_sh, deg_sh, srcbig, dstbig, rowsA, rowsB, ones_v, zbuf,
              degb, semA, semB):
    c = lax.axis_index("c")
    s = lax.axis_index("s")
    wid = s * _NC + c
    r0 = s * _RPT

    def _init_zrow(i, carry):
        for j in range(_D // 16):
            zbuf[i, pl.ds(j * 16, 16)] = jnp.zeros((16,), jnp.float32)
        return carry
    lax.fori_loop(0, _HB, _init_zrow, 0)

    def _init_ones(j, carry):
        ones_v[pl.ds(j * 16, 16)] = jnp.ones((16,), jnp.float32)
        return carry
    lax.fori_loop(0, _K // 16, _init_ones, 0)
    _zero_vec(degb, _RPT)

    for blk in range(_RPT // _HB):
        pltpu.sync_copy(zbuf, agg_sh.at[pl.ds(r0 + blk * _HB, _HB), :])
    pltpu.sync_copy(degb, deg_sh.at[pl.ds(r0, _RPT)])
    plsc.subcore_barrier()

    for db in range(_CD // _IB):
        base = s * _CD + db * _IB
        pltpu.sync_copy(dst_h.at[pl.ds(base, _IB), :], dstbig)
        dst2 = dstbig
        for t in range(_IB):
            pltpu.sync_copy(ones_v, deg_sh.at[dst2.at[t]], add=True)

    _gather_scatter_edges(tab_h, src_h, dst_h, agg_sh,
                          srcbig, dstbig, rowsA, rowsB, semA, semB, wid)

    plsc.subcore_barrier()

    pltpu.sync_copy(deg_sh.at[pl.ds(r0, _RPT)], degb)

    def _inv(j, carry):
        dv = degb[pl.ds(j * 16, 16)]
        degb[pl.ds(j * 16, 16)] = 1.0 / jnp.maximum(dv, 1.0)
        return carry
    lax.fori_loop(0, _RPT // 16, _inv, 0)

    @pl.when(c == 0)
    def _():
        pltpu.sync_copy(degb, invdeg_h.at[pl.ds(r0, _RPT)])

    for blk in range(_RPT // _HB):
        rb = r0 + blk * _HB
        pltpu.sync_copy(agg_sh.at[pl.ds(rb, _HB), :], zbuf)

        def _norm_rows(t, carry, blk=blk):
            inv = degb[pl.ds(blk * _HB + t * 16, 16)]
            for i in range(16):
                r = t * 16 + i
                iv = inv[i]
                for j in range(_D // 16):
                    zbuf[r, pl.ds(j * 16, 16)] = (
                        zbuf[r, pl.ds(j * 16, 16)] * iv)
            return carry
        lax.fori_loop(0, _HB // 16, _norm_rows, 0)

        pltpu.sync_copy(zbuf, out_h.at[c, pl.ds(rb, _HB), :])


def _mp2_body(tab_h, src_h, dst_h, invdeg_h, out_h,
              agg_sh, srcbig, dstbig, rowsA, rowsB, zbuf, degb, semA, semB):
    c = lax.axis_index("c")
    s = lax.axis_index("s")
    wid = s * _NC + c
    r0 = s * _RPT

    def _init_zrow(i, carry):
        for j in range(_D // 16):
            zbuf[i, pl.ds(j * 16, 16)] = jnp.zeros((16,), jnp.float32)
        return carry
    lax.fori_loop(0, _HB, _init_zrow, 0)

    for blk in range(_RPT // _HB):
        pltpu.sync_copy(zbuf, agg_sh.at[pl.ds(r0 + blk * _HB, _HB), :])
    plsc.subcore_barrier()

    _gather_scatter_edges(tab_h, src_h, dst_h, agg_sh,
                          srcbig, dstbig, rowsA, rowsB, semA, semB, wid)

    plsc.subcore_barrier()

    pltpu.sync_copy(invdeg_h.at[pl.ds(r0, _RPT)], degb)
    for blk in range(_RPT // _HB):
        rb = r0 + blk * _HB
        pltpu.sync_copy(agg_sh.at[pl.ds(rb, _HB), :], zbuf)

        def _norm_rows(t, carry, blk=blk):
            inv = degb[pl.ds(blk * _HB + t * 16, 16)]
            for i in range(16):
                r = t * 16 + i
                iv = inv[i]
                for j in range(_D // 16):
                    zbuf[r, pl.ds(j * 16, 16)] = (
                        zbuf[r, pl.ds(j * 16, 16)] * iv)
            return carry
        lax.fori_loop(0, _HB // 16, _norm_rows, 0)

        pltpu.sync_copy(zbuf, out_h.at[c, pl.ds(rb, _HB), :])


_sc_mesh = plsc.VectorSubcoreMesh(core_axis_name="c", subcore_axis_name="s")

_mp1_call = pl.kernel(
    _mp1_body,
    out_type=(jax.ShapeDtypeStruct((_NC, _NPAD, _D), jnp.float32),
              jax.ShapeDtypeStruct((_NPAD,), jnp.float32)),
    mesh=_sc_mesh,
    scratch_types=[
        pltpu.VMEM_SHARED((_NPAD, _D), jnp.float32),
        pltpu.VMEM_SHARED((_NPAD,), jnp.float32),
        pltpu.VMEM((_IB, _K), jnp.int32),
        pltpu.VMEM((_IB, _K), jnp.int32),
        pltpu.VMEM((_K, _D), jnp.float32),
        pltpu.VMEM((_K, _D), jnp.float32),
        pltpu.VMEM((_K,), jnp.float32),
        pltpu.VMEM((_HB, _D), jnp.float32),
        pltpu.VMEM((_RPT,), jnp.float32),
        pltpu.SemaphoreType.DMA,
        pltpu.SemaphoreType.DMA,
    ],
)

_mp2_call = pl.kernel(
    _mp2_body,
    out_type=jax.ShapeDtypeStruct((_NC, _NPAD, _D), jnp.float32),
    mesh=_sc_mesh,
    scratch_types=[
        pltpu.VMEM_SHARED((_NPAD, _D), jnp.float32),
        pltpu.VMEM((_IB, _K), jnp.int32),
        pltpu.VMEM((_IB, _K), jnp.int32),
        pltpu.VMEM((_K, _D), jnp.float32),
        pltpu.VMEM((_K, _D), jnp.float32),
        pltpu.VMEM((_HB, _D), jnp.float32),
        pltpu.VMEM((_RPT,), jnp.float32),
        pltpu.SemaphoreType.DMA,
        pltpu.SemaphoreType.DMA,
    ],
)


def _tc1_body(p_ref, w_ref, b_ref, g_ref, be_ref, a_ref, o_ref):
    h = p_ref[0, :_N, :] + p_ref[1, :_N, :]
    h = jnp.dot(h, w_ref[...], preferred_element_type=jnp.float32) + b_ref[...]
    m = jnp.mean(h, axis=0, keepdims=True)
    v = jnp.mean((h - m) * (h - m), axis=0, keepdims=True)
    h = (h - m) * lax.rsqrt(v + 1e-5) * g_ref[...] + be_ref[...]
    a = a_ref[0, 0]
    o_ref[...] = jnp.where(h > 0, h, a * h)


_tc1_call = pl.pallas_call(
    _tc1_body,
    out_shape=jax.ShapeDtypeStruct((_N, _D), jnp.float32),
)


def _tc2_body(p_ref, w_ref, b_ref, g_ref, be_ref, a_ref, wc_ref, bc_ref,
              o_ref):
    h = p_ref[0, :_N, :] + p_ref[1, :_N, :]
    h = jnp.dot(h, w_ref[...], preferred_element_type=jnp.float32) + b_ref[...]
    m = jnp.mean(h, axis=0, keepdims=True)
    v = jnp.mean((h - m) * (h - m), axis=0, keepdims=True)
    h = (h - m) * lax.rsqrt(v + 1e-5) * g_ref[...] + be_ref[...]
    a = a_ref[0, 0]
    h = jnp.where(h > 0, h, a * h)
    lg = jnp.dot(h, wc_ref[...], preferred_element_type=jnp.float32)
    lg = lg + bc_ref[...]
    mx = jnp.max(lg, axis=-1, keepdims=True)
    e = jnp.exp(lg - mx)
    o_ref[...] = e / jnp.sum(e, axis=-1, keepdims=True) + 1e-10


_tc2_call = pl.pallas_call(
    _tc2_body,
    out_shape=jax.ShapeDtypeStruct((_N, _NCLS), jnp.float32),
)


def kernel(x, edge_index, W1, b1, g1, be1, a1, W2, b2, g2, be2, a2, Wc, bc):
    src = edge_index[0].astype(jnp.int32)
    dst = edge_index[1].astype(jnp.int32)
    pad = _EPAD - _E
    srcp = jnp.concatenate([src, jnp.zeros((pad,), jnp.int32)])
    dstp = jnp.concatenate([dst, jnp.full((pad,), _N, jnp.int32)])
    srcp = srcp.reshape(_EPAD // _K, _K)
    dstp = dstp.reshape(_EPAD // _K, _K)

    p1, invdeg = _mp1_call(x, srcp, dstp)
    h1 = _tc1_call(p1, W1, b1.reshape(1, _D), g1.reshape(1, _D),
                   be1.reshape(1, _D), a1.reshape(1, 1))
    p2 = _mp2_call(h1, srcp, dstp, invdeg)
    return _tc2_call(p2, W2, b2.reshape(1, _D), g2.reshape(1, _D),
                     be2.reshape(1, _D), a2.reshape(1, 1),
                     Wc, bc.reshape(1, _NCLS))


# speedup vs baseline: 10.5055x; 3.0554x over previous
import jax
import jax.numpy as jnp
from jax import lax
from jax.experimental import pallas as pl
from jax.experimental.pallas import tpu as pltpu
from jax.experimental.pallas import tpu_sc as plsc

_N = 10000
_D = 128
_E = 320000
_NCLS = 40

_NC = 2
_NS = 16
_NW = _NC * _NS
_K = 128
_IB = 16
_CW = 80
_EPAD = _NW * _CW * _K
_CD = _EPAD // (_NS * _K)
_NPAD = 10240
_RPT = _NPAD // _NS
_HB = 64


def _zero_vec(ref, n):
    def _z(j, carry):
        ref[pl.ds(j * 16, 16)] = jnp.zeros((16,), jnp.float32)
        return carry
    lax.fori_loop(0, n // 16, _z, 0)


def _gather_scatter_edges(tab_h, src_h, dst_h, agg_sh,
                          srcbig, dstbig, rowsA, rowsB, semA, semB, wid):
    for bi in range(_CW // _IB):
        base = wid * _CW + bi * _IB
        pltpu.sync_copy(src_h.at[pl.ds(base, _IB), :], srcbig)
        pltpu.sync_copy(dst_h.at[pl.ds(base, _IB), :], dstbig)
        src2 = srcbig
        dst2 = dstbig
        pending = None
        for t in range(_IB):
            buf, sem = (rowsA, semA) if t % 2 == 0 else (rowsB, semB)
            cp = pltpu.async_copy(tab_h.at[src2.at[t]], buf, sem)
            if pending is not None:
                pending[0].wait()
                pltpu.sync_copy(pending[1], agg_sh.at[dst2.at[pending[2]]],
                                add=True)
            pending = (cp, buf, t)
        pending[0].wait()
        pltpu.sync_copy(pending[1], agg_sh.at[dst2.at[pending[2]]], add=True)


def _mp1_body(tab_h, src_h, dst_h, out_h, invdeg_h,
              agg_sh, deg_sh, srcbig, dstbig, rowsA, rowsB, ones_v, zbuf,
              degb, semA, semB):
    c = lax.axis_index("c")
    s = lax.axis_index("s")
    wid = s * _NC + c
    r0 = s * _RPT

    def _init_zrow(i, carry):
        for j in range(_D // 16):
            zbuf[i, pl.ds(j * 16, 16)] = jnp.zeros((16,), jnp.float32)
        return carry
    lax.fori_loop(0, _HB, _init_zrow, 0)

    def _init_ones(j, carry):
        ones_v[pl.ds(j * 16, 16)] = jnp.ones((16,), jnp.float32)
        return carry
    lax.fori_loop(0, _K // 16, _init_ones, 0)
    _zero_vec(degb, _RPT)

    for blk in range(_RPT // _HB):
        pltpu.sync_copy(zbuf, agg_sh.at[pl.ds(r0 + blk * _HB, _HB), :])
    pltpu.sync_copy(degb, deg_sh.at[pl.ds(r0, _RPT)])
    plsc.subcore_barrier()

    for db in range(_CD // _IB):
        base = s * _CD + db * _IB
        pltpu.sync_copy(dst_h.at[pl.ds(base, _IB), :], dstbig)
        dst2 = dstbig
        for t in range(_IB):
            pltpu.sync_copy(ones_v, deg_sh.at[dst2.at[t]], add=True)

    _gather_scatter_edges(tab_h, src_h, dst_h, agg_sh,
                          srcbig, dstbig, rowsA, rowsB, semA, semB, wid)

    plsc.subcore_barrier()

    pltpu.sync_copy(deg_sh.at[pl.ds(r0, _RPT)], degb)

    def _inv(j, carry):
        dv = degb[pl.ds(j * 16, 16)]
        degb[pl.ds(j * 16, 16)] = 1.0 / jnp.maximum(dv, 1.0)
        return carry
    lax.fori_loop(0, _RPT // 16, _inv, 0)

    @pl.when(c == 0)
    def _():
        pltpu.sync_copy(degb, invdeg_h.at[pl.ds(r0, _RPT)])

    for blk in range(_RPT // _HB):
        rb = r0 + blk * _HB
        pltpu.sync_copy(agg_sh.at[pl.ds(rb, _HB), :], zbuf)

        def _norm_rows(t, carry, blk=blk):
            inv = degb[pl.ds(blk * _HB + t * 16, 16)]
            for i in range(16):
                r = t * 16 + i
                iv = inv[i]
                for j in range(_D // 16):
                    zbuf[r, pl.ds(j * 16, 16)] = (
                        zbuf[r, pl.ds(j * 16, 16)] * iv)
            return carry
        lax.fori_loop(0, _HB // 16, _norm_rows, 0)

        pltpu.sync_copy(zbuf, out_h.at[c, pl.ds(rb, _HB), :])


def _mp2_body(tab_h, src_h, dst_h, invdeg_h, out_h,
              agg_sh, srcbig, dstbig, rowsA, rowsB, zbuf, degb, semA, semB):
    c = lax.axis_index("c")
    s = lax.axis_index("s")
    wid = s * _NC + c
    r0 = s * _RPT

    def _init_zrow(i, carry):
        for j in range(_D // 16):
            zbuf[i, pl.ds(j * 16, 16)] = jnp.zeros((16,), jnp.float32)
        return carry
    lax.fori_loop(0, _HB, _init_zrow, 0)

    for blk in range(_RPT // _HB):
        pltpu.sync_copy(zbuf, agg_sh.at[pl.ds(r0 + blk * _HB, _HB), :])
    plsc.subcore_barrier()

    _gather_scatter_edges(tab_h, src_h, dst_h, agg_sh,
                          srcbig, dstbig, rowsA, rowsB, semA, semB, wid)

    plsc.subcore_barrier()

    pltpu.sync_copy(invdeg_h.at[pl.ds(r0, _RPT)], degb)
    for blk in range(_RPT // _HB):
        rb = r0 + blk * _HB
        pltpu.sync_copy(agg_sh.at[pl.ds(rb, _HB), :], zbuf)

        def _norm_rows(t, carry, blk=blk):
            inv = degb[pl.ds(blk * _HB + t * 16, 16)]
            for i in range(16):
                r = t * 16 + i
                iv = inv[i]
                for j in range(_D // 16):
                    zbuf[r, pl.ds(j * 16, 16)] = (
                        zbuf[r, pl.ds(j * 16, 16)] * iv)
            return carry
        lax.fori_loop(0, _HB // 16, _norm_rows, 0)

        pltpu.sync_copy(zbuf, out_h.at[c, pl.ds(rb, _HB), :])


_sc_mesh = plsc.VectorSubcoreMesh(core_axis_name="c", subcore_axis_name="s")

_mp1_call = pl.kernel(
    _mp1_body,
    out_type=(jax.ShapeDtypeStruct((_NC, _NPAD, _D), jnp.float32),
              jax.ShapeDtypeStruct((_NPAD,), jnp.float32)),
    mesh=_sc_mesh,
    scratch_types=[
        pltpu.VMEM_SHARED((_NPAD, _D), jnp.float32),
        pltpu.VMEM_SHARED((_NPAD,), jnp.float32),
        pltpu.VMEM((_IB, _K), jnp.int32),
        pltpu.VMEM((_IB, _K), jnp.int32),
        pltpu.VMEM((_K, _D), jnp.float32),
        pltpu.VMEM((_K, _D), jnp.float32),
        pltpu.VMEM((_K,), jnp.float32),
        pltpu.VMEM((_HB, _D), jnp.float32),
        pltpu.VMEM((_RPT,), jnp.float32),
        pltpu.SemaphoreType.DMA,
        pltpu.SemaphoreType.DMA,
    ],
)

_mp2_call = pl.kernel(
    _mp2_body,
    out_type=jax.ShapeDtypeStruct((_NC, _NPAD, _D), jnp.float32),
    mesh=_sc_mesh,
    scratch_types=[
        pltpu.VMEM_SHARED((_NPAD, _D), jnp.float32),
        pltpu.VMEM((_IB, _K), jnp.int32),
        pltpu.VMEM((_IB, _K), jnp.int32),
        pltpu.VMEM((_K, _D), jnp.float32),
        pltpu.VMEM((_K, _D), jnp.float32),
        pltpu.VMEM((_HB, _D), jnp.float32),
        pltpu.VMEM((_RPT,), jnp.float32),
        pltpu.SemaphoreType.DMA,
        pltpu.SemaphoreType.DMA,
    ],
)


def _tc1_body(p_ref, w_ref, b_ref, g_ref, be_ref, a_ref, o_ref):
    h = p_ref[0, :_N, :] + p_ref[1, :_N, :]
    h = jnp.dot(h, w_ref[...], preferred_element_type=jnp.float32) + b_ref[...]
    m = jnp.mean(h, axis=0, keepdims=True)
    v = jnp.mean((h - m) * (h - m), axis=0, keepdims=True)
    h = (h - m) * lax.rsqrt(v + 1e-5) * g_ref[...] + be_ref[...]
    a = a_ref[0, 0]
    o_ref[...] = jnp.where(h > 0, h, a * h)


_tc1_call = pl.pallas_call(
    _tc1_body,
    out_shape=jax.ShapeDtypeStruct((_N, _D), jnp.float32),
)


def _tc2_body(p_ref, w_ref, b_ref, g_ref, be_ref, a_ref, wc_ref, bc_ref,
              o_ref):
    h = p_ref[0, :_N, :] + p_ref[1, :_N, :]
    h = jnp.dot(h, w_ref[...], preferred_element_type=jnp.float32) + b_ref[...]
    m = jnp.mean(h, axis=0, keepdims=True)
    v = jnp.mean((h - m) * (h - m), axis=0, keepdims=True)
    h = (h - m) * lax.rsqrt(v + 1e-5) * g_ref[...] + be_ref[...]
    a = a_ref[0, 0]
    h = jnp.where(h > 0, h, a * h)
    lg = jnp.dot(h, wc_ref[...], preferred_element_type=jnp.float32)
    lg = lg + bc_ref[...]
    mx = jnp.max(lg, axis=-1, keepdims=True)
    e = jnp.exp(lg - mx)
    o_ref[...] = e / jnp.sum(e, axis=-1, keepdims=True) + 1e-10


_tc2_call = pl.pallas_call(
    _tc2_body,
    out_shape=jax.ShapeDtypeStruct((_N, _NCLS), jnp.float32),
)


def kernel(x, edge_index, W1, b1, g1, be1, a1, W2, b2, g2, be2, a2, Wc, bc):
    src = edge_index[0].astype(jnp.int32)
    dst = edge_index[1].astype(jnp.int32)
    pad = _EPAD - _E
    padidx = jax.lax.iota(jnp.int32, pad)
    srcp = jnp.concatenate([src, padidx % _N])
    dstp = jnp.concatenate([dst, _N + padidx % (_NPAD - _N)])
    srcp = srcp.reshape(_EPAD // _K, _K)
    dstp = dstp.reshape(_EPAD // _K, _K)

    p1, invdeg = _mp1_call(x, srcp, dstp)
    h1 = _tc1_call(p1, W1, b1.reshape(1, _D), g1.reshape(1, _D),
                   be1.reshape(1, _D), a1.reshape(1, 1))
    p2 = _mp2_call(h1, srcp, dstp, invdeg)
    return _tc2_call(p2, W2, b2.reshape(1, _D), g2.reshape(1, _D),
                     be2.reshape(1, _D), a2.reshape(1, 1),
                     Wc, bc.reshape(1, _NCLS))
